# Initial kernel scaffold; baseline (speedup 1.0000x reference)
import jax
import jax.numpy as jnp
from jax.experimental import pallas as pl


def kernel(x, mat_rows, mat_cols, mat_vals, mask):
    raise NotImplementedError("write your pallas kernel here")


# speedup vs baseline: 2.1509x; 2.1509x over previous
import jax
import jax.numpy as jnp
from jax import lax
from jax.experimental import pallas as pl
from jax.experimental.pallas import tpu as pltpu
from jax.experimental.pallas import tpu_sc as plsc

B = 16
C = 3
TEX = 512
IMG = 224
NNZ = 602112
N_TEX = TEX * TEX * C
N_OUT = IMG * IMG * C

NC = 2
NS = 16
HALF = N_OUT // NC
TRASH = HALF
ACC_ROWS = HALF + 8
PER_TILE = NNZ // NS
SUB = 128
NSUB = PER_TILE // SUB
NBUF = 3
OUTER = NSUB // NBUF
ZROWS = HALF // NS // 4
STRIPE = HALF // NS


def _body(x_hbm, rows_hbm, cols_hbm, vals_hbm, out_hbm,
          rbuf, cbuf, vbuf, lrow, xbuf, ybuf, zbuf, acc,
          ss0, ss1, ss2, gs0, gs1, gs2, cs0, cs1, cs2):
    ssem = (ss0, ss1, ss2)
    gsem = (gs0, gs1, gs2)
    csem = (cs0, cs1, cs2)
    c = lax.axis_index("c")
    s = lax.axis_index("s")
    base_nnz = s * PER_TILE
    base_row = c * HALF

    @pl.loop(0, ZROWS, unroll=8)
    def _zero(j):
        zbuf[j, :] = jnp.zeros((16,), jnp.float32)

    for k in range(4):
        pltpu.sync_copy(zbuf, acc.at[pl.ds(s * STRIPE + k * ZROWS, ZROWS)])
    @pl.when(s == NS - 1)
    def _zero_pad():
        pltpu.sync_copy(zbuf.at[pl.ds(0, 8)], acc.at[pl.ds(HALF, 8)])

    plsc.subcore_barrier()

    def issue_stage(g, b):
        off = base_nnz + g * SUB
        pltpu.async_copy(rows_hbm.at[pl.ds(off, SUB)], rbuf.at[b], ssem[b])
        pltpu.async_copy(cols_hbm.at[pl.ds(off, SUB)], cbuf.at[b], ssem[b])
        pltpu.async_copy(vals_hbm.at[pl.ds(off, SUB)], vbuf.at[b], ssem[b])

    def wait_stage(b):
        pltpu.make_async_copy(rows_hbm.at[pl.ds(0, SUB)], rbuf.at[b], ssem[b]).wait()
        pltpu.make_async_copy(cols_hbm.at[pl.ds(0, SUB)], cbuf.at[b], ssem[b]).wait()
        pltpu.make_async_copy(vals_hbm.at[pl.ds(0, SUB)], vbuf.at[b], ssem[b]).wait()

    def issue_gather(b):
        pltpu.async_copy(x_hbm.at[cbuf.at[b]], xbuf.at[b], gsem[b])

    def wait_gather(b):
        pltpu.make_async_copy(x_hbm.at[pl.ds(0, SUB)], xbuf.at[b], gsem[b]).wait()

    def issue_scatter(b):
        pltpu.async_copy(ybuf.at[b], acc.at[lrow.at[b]], csem[b], add=True)

    def wait_scatter(b):
        pltpu.make_async_copy(ybuf.at[b], acc.at[pl.ds(0, SUB)], csem[b]).wait()

    for b in range(NBUF):
        issue_stage(b, b)
    for b in range(2):
        wait_stage(b)
        issue_gather(b)

    @pl.loop(0, OUTER)
    def _outer(outer):
        for b in range(NBUF):
            g = outer * NBUF + b

            wait_gather(b)

            @pl.when(outer > 0)
            def _drain():
                wait_scatter(b)

            for j8 in range(SUB // 16):
                r = rbuf[b, pl.ds(j8 * 16, 16)]
                lr = r - base_row
                ok = (lr >= 0) & (lr < HALF)
                lrow[b, pl.ds(j8 * 16, 16)] = jnp.where(ok, lr, TRASH)

            @pl.loop(0, SUB // 16)
            def _mul(j16):
                jb = j16 * 16
                vv = vbuf[b, pl.ds(jb, 16)]
                for j in range(16):
                    ybuf[b, jb + j, :] = xbuf[b, jb + j, :] * vv[j]

            issue_scatter(b)

            @pl.when(g + 2 < NSUB)
            def _next_gather():
                b2 = (b + 2) % NBUF
                wait_stage(b2)
                issue_gather(b2)

            @pl.when(g + NBUF < NSUB)
            def _next_stage():
                issue_stage(g + NBUF, b)

    for b in range(NBUF):
        wait_scatter(b)

    plsc.subcore_barrier()

    pltpu.sync_copy(acc.at[pl.ds(s * STRIPE, STRIPE)],
                    out_hbm.at[pl.ds(base_row + s * STRIPE, STRIPE)])


def _spmm(x_flat, mat_rows, mat_cols, mat_vals):
    mesh = plsc.VectorSubcoreMesh(core_axis_name="c", subcore_axis_name="s",
                                  num_cores=NC, num_subcores=NS)
    f = pl.kernel(
        _body,
        out_type=jax.ShapeDtypeStruct((N_OUT, B), jnp.float32),
        mesh=mesh,
        scratch_types=[
            pltpu.VMEM((NBUF, SUB), jnp.int32),
            pltpu.VMEM((NBUF, SUB), jnp.int32),
            pltpu.VMEM((NBUF, SUB), jnp.float32),
            pltpu.VMEM((NBUF, SUB), jnp.int32),
            pltpu.VMEM((NBUF, SUB, B), jnp.float32),
            pltpu.VMEM((NBUF, SUB, B), jnp.float32),
            pltpu.VMEM((ZROWS, B), jnp.float32),
            pltpu.VMEM_SHARED((ACC_ROWS, B), jnp.float32),
        ] + [pltpu.SemaphoreType.DMA] * 9,
        compiler_params=pltpu.CompilerParams(use_tc_tiling_on_sc=False),
        name="coo_spmm_sc",
    )
    return f(x_flat, mat_rows, mat_cols, mat_vals)


@jax.jit
def kernel(x, mat_rows, mat_cols, mat_vals, mask):
    b = x.shape[0]
    x_flat = jnp.transpose(x, (0, 2, 3, 1)).reshape(b, -1).T
    out_flat = _spmm(x_flat, mat_rows, mat_cols, mat_vals)
    result = out_flat.T.reshape(b, IMG, IMG, C)
    result = jnp.transpose(result, (0, 3, 1, 2))
    bbox = ((48, 80), (112, 208))
    return (result, mask, jnp.asarray(bbox))
